# baseline (speedup 1.0000x reference)
import jax
import jax.numpy as jnp
from jax.experimental import pallas as pl
from jax.experimental.pallas import tpu as pltpu

_LD = 512
_NCB = 32
_K = 1024
_D = 32
_TB = 1024
_NC = 2
_CH = _TB // _NC


def _vq_body(lat_ref, w_ref, b_ref, cb_ref, q_ref, idx_ref):
    lat = lat_ref[0]
    w = w_ref[...]
    x = jax.lax.dot_general(
        lat.astype(jnp.bfloat16), w.astype(jnp.bfloat16),
        (((0,), (1,)), ((), ())),
        preferred_element_type=jnp.float32)
    x = x + b_ref[...]

    iota = jax.lax.broadcasted_iota(jnp.int32, (_CH, _K), 1)
    one = jnp.ones((), jnp.bfloat16)
    zero = jnp.zeros((), jnp.bfloat16)
    res = [x[c * _CH:(c + 1) * _CH, :] for c in range(_NC)]
    quant = [jnp.zeros((_CH, _D), jnp.float32) for _ in range(_NC)]
    for i in range(_NCB):
        cb = cb_ref[i]
        cn = jnp.sum(cb * cb, axis=1, keepdims=True)
        cn_row = cn.reshape(1, _K)
        cbb = cb.astype(jnp.bfloat16)
        c1f = cbb.astype(jnp.float32)
        c2 = (cb - c1f).astype(jnp.bfloat16)
        c2f = c2.astype(jnp.float32)
        c3 = (cb - c1f - c2f).astype(jnp.bfloat16)
        cpack = jnp.concatenate([cbb, c2, c3], axis=1)
        for c in range(_NC):
            an = jnp.sum(res[c] * res[c], axis=1, keepdims=True)
            dot = jax.lax.dot_general(
                res[c].astype(jnp.bfloat16), cbb,
                (((1,), (1,)), ((), ())),
                preferred_element_type=jnp.float32)
            d2 = (an - 2.0 * dot) + cn_row
            m2 = jnp.maximum(jnp.min(d2, axis=1, keepdims=True), 0.0)
            sm = jnp.sqrt(m2)
            ulp = jax.lax.bitcast_convert_type(
                jax.lax.bitcast_convert_type(sm, jnp.int32) + 1,
                jnp.float32) - sm
            thr = sm * sm + sm * ulp
            thr = jnp.maximum(thr, m2)
            cand = jnp.where(d2 <= thr, iota, _K)
            hard = jnp.min(cand, axis=1, keepdims=True)
            idx_ref[0, c * _CH:(c + 1) * _CH, i:i + 1] = hard
            onehot = (iota == hard).astype(jnp.bfloat16)
            sp = jax.lax.dot_general(
                onehot, cpack, (((1,), (0,)), ((), ())),
                preferred_element_type=jnp.float32)
            step = (sp[:, 0:_D] + sp[:, _D:2 * _D]) + sp[:, 2 * _D:3 * _D]
            quant[c] = quant[c] + step
            res[c] = res[c] - step
    qc = jnp.concatenate(quant, axis=0)
    q_ref[0] = x + (qc - x)


def kernel(latent, W, b, codebooks):
    B, _, T = latent.shape
    grid = (B, T // _TB)
    qf, idxf = pl.pallas_call(
        _vq_body,
        grid=grid,
        in_specs=[
            pl.BlockSpec((1, _LD, _TB), lambda bi, ti: (bi, 0, ti)),
            pl.BlockSpec((_D, _LD), lambda bi, ti: (0, 0)),
            pl.BlockSpec((1, _D), lambda bi, ti: (0, 0)),
            pl.BlockSpec((_NCB, _K, _D), lambda bi, ti: (0, 0, 0)),
        ],
        out_specs=[
            pl.BlockSpec((1, _TB, _D), lambda bi, ti: (bi, ti, 0)),
            pl.BlockSpec((1, _TB, _NCB), lambda bi, ti: (bi, ti, 0)),
        ],
        out_shape=[
            jax.ShapeDtypeStruct((B, T, _D), jnp.float32),
            jax.ShapeDtypeStruct((B, T, _NCB), jnp.int32),
        ],
        compiler_params=pltpu.CompilerParams(
            dimension_semantics=("parallel", "parallel"),
            vmem_limit_bytes=100 * 1024 * 1024),
    )(latent, W, b.reshape(1, _D), codebooks)
    quant = jnp.transpose(qf, (0, 2, 1))
    idxs = jnp.transpose(idxf, (2, 0, 1))
    return quant, idxs


# speedup vs baseline: 1.4264x; 1.4264x over previous
import jax
import jax.numpy as jnp
from jax.experimental import pallas as pl
from jax.experimental.pallas import tpu as pltpu

_LD = 512
_NCB = 32
_K = 1024
_D = 32
_TB = 1024
_NC = 1
_CH = _TB // _NC


def _vq_body(lat_ref, w_ref, b_ref, cb_ref, q_ref, idx_ref):
    lat = lat_ref[0]
    w = w_ref[...]
    x = jax.lax.dot_general(
        lat.astype(jnp.bfloat16), w.astype(jnp.bfloat16),
        (((0,), (1,)), ((), ())),
        preferred_element_type=jnp.float32)
    x = x + b_ref[...]

    iota = jax.lax.broadcasted_iota(jnp.int32, (_CH, _K), 1)
    one = jnp.ones((), jnp.bfloat16)
    zero = jnp.zeros((), jnp.bfloat16)
    res = [x[c * _CH:(c + 1) * _CH, :] for c in range(_NC)]
    quant = [jnp.zeros((_CH, _D), jnp.float32) for _ in range(_NC)]
    for i in range(_NCB):
        cb = cb_ref[i]
        cn = jnp.sum(cb * cb, axis=1, keepdims=True)
        cn_row = cn.reshape(1, _K)
        cbb = cb.astype(jnp.bfloat16)
        c1f = cbb.astype(jnp.float32)
        c2 = (cb - c1f).astype(jnp.bfloat16)
        c2f = c2.astype(jnp.float32)
        c3 = (cb - c1f - c2f).astype(jnp.bfloat16)
        cpack = jnp.concatenate([cbb, c2, c3], axis=1)
        for c in range(_NC):
            an = jnp.sum(res[c] * res[c], axis=1, keepdims=True)
            dot = jax.lax.dot_general(
                res[c].astype(jnp.bfloat16), cbb,
                (((1,), (1,)), ((), ())),
                preferred_element_type=jnp.float32)
            d2 = (an - 2.0 * dot) + cn_row
            m2 = jnp.maximum(jnp.min(d2, axis=1, keepdims=True), 0.0)
            sm = jnp.sqrt(m2)
            ulp = jax.lax.bitcast_convert_type(
                jax.lax.bitcast_convert_type(sm, jnp.int32) + 1,
                jnp.float32) - sm
            thr = sm * sm + sm * ulp
            thr = jnp.maximum(thr, m2)
            cand = jnp.where(d2 <= thr, iota, _K)
            hard = jnp.min(cand, axis=1, keepdims=True)
            idx_ref[0, c * _CH:(c + 1) * _CH, i:i + 1] = hard
            onehot = (iota == hard).astype(jnp.bfloat16)
            sp = jax.lax.dot_general(
                onehot, cpack, (((1,), (0,)), ((), ())),
                preferred_element_type=jnp.float32)
            step = (sp[:, 0:_D] + sp[:, _D:2 * _D]) + sp[:, 2 * _D:3 * _D]
            quant[c] = quant[c] + step
            res[c] = res[c] - step
    qc = jnp.concatenate(quant, axis=0)
    q_ref[0] = x + (qc - x)


def kernel(latent, W, b, codebooks):
    B, _, T = latent.shape
    grid = (B, T // _TB)
    qf, idxf = pl.pallas_call(
        _vq_body,
        grid=grid,
        in_specs=[
            pl.BlockSpec((1, _LD, _TB), lambda bi, ti: (bi, 0, ti)),
            pl.BlockSpec((_D, _LD), lambda bi, ti: (0, 0)),
            pl.BlockSpec((1, _D), lambda bi, ti: (0, 0)),
            pl.BlockSpec((_NCB, _K, _D), lambda bi, ti: (0, 0, 0)),
        ],
        out_specs=[
            pl.BlockSpec((1, _TB, _D), lambda bi, ti: (bi, ti, 0)),
            pl.BlockSpec((1, _TB, _NCB), lambda bi, ti: (bi, ti, 0)),
        ],
        out_shape=[
            jax.ShapeDtypeStruct((B, T, _D), jnp.float32),
            jax.ShapeDtypeStruct((B, T, _NCB), jnp.int32),
        ],
        compiler_params=pltpu.CompilerParams(
            dimension_semantics=("parallel", "parallel"),
            vmem_limit_bytes=100 * 1024 * 1024),
    )(latent, W, b.reshape(1, _D), codebooks)
    quant = jnp.transpose(qf, (0, 2, 1))
    idxs = jnp.transpose(idxf, (2, 0, 1))
    return quant, idxs
